# baseline (speedup 1.0000x reference)
import functools

import jax
import jax.numpy as jnp
from jax import lax
from jax.experimental import pallas as pl
from jax.experimental.pallas import tpu as pltpu
from jax.experimental.pallas import tpu_sc as plsc

SEQ = 2048
DIM = 768
DFF = 512
NE = 8
TOPK = 2
TB = 128
NBLK = 40
CAP = NBLK * TB
NW = 32
TPW = SEQ // NW
_SUB = 256


def _routing_body(logits_ref, sel_ref, rws0_ref, rws1_ref,
                  pos0_ref, pos1_ref, be_ref):
    l = logits_ref[...]
    ii = lax.broadcasted_iota(jnp.int32, (SEQ, NE), 1)
    m0 = jnp.max(l, axis=1, keepdims=True)
    e0 = jnp.min(jnp.where(l == m0, ii, NE), axis=1, keepdims=True)
    lmask = jnp.where(ii == e0, -jnp.inf, l)
    m1 = jnp.max(lmask, axis=1, keepdims=True)
    e1 = jnp.min(jnp.where(lmask == m1, ii, NE), axis=1, keepdims=True)
    b = jnp.exp(m1 - m0)
    rw0 = 1.0 / (1.0 + b)
    rw1 = b / (1.0 + b)
    sel_ref[...] = jnp.concatenate([e0, e1], axis=1)
    rws0_ref[...] = jnp.broadcast_to(rw0, (SEQ, 16))
    rws1_ref[...] = jnp.broadcast_to(rw1, (SEQ, 16))

    oh0 = (jnp.broadcast_to(e0, (SEQ, NE)) == ii).astype(jnp.float32)
    oh1 = (jnp.broadcast_to(e1, (SEQ, NE)) == ii).astype(jnp.float32)
    nb = SEQ // _SUB
    r_i = lax.broadcasted_iota(jnp.int32, (_SUB, _SUB), 0)
    c_i = lax.broadcasted_iota(jnp.int32, (_SUB, _SUB), 1)
    ltri = (c_i < r_i).astype(jnp.float32)
    cw0s, cw1s, cb0s, cb1s = [], [], [], []
    cb0 = jnp.zeros((1, NE), jnp.float32)
    cb1 = jnp.zeros((1, NE), jnp.float32)
    for b in range(nb):
        o0b = oh0[b * _SUB:(b + 1) * _SUB, :]
        o1b = oh1[b * _SUB:(b + 1) * _SUB, :]
        cw0s.append(lax.dot_general(ltri, o0b, (((1,), (0,)), ((), ())),
                                    preferred_element_type=jnp.float32))
        cw1s.append(lax.dot_general(ltri, o1b, (((1,), (0,)), ((), ())),
                                    preferred_element_type=jnp.float32))
        cb0s.append(cb0)
        cb1s.append(cb1)
        cb0 = cb0 + jnp.sum(o0b, axis=0, keepdims=True)
        cb1 = cb1 + jnp.sum(o1b, axis=0, keepdims=True)
    cnt0 = cb0
    cnt = cb0 + cb1
    pc = jnp.ceil(cnt * (1.0 / TB)) * TB
    r8 = lax.broadcasted_iota(jnp.int32, (NE, NE), 0)
    c8 = lax.broadcasted_iota(jnp.int32, (NE, NE), 1)
    l8 = (r8 < c8).astype(jnp.float32)
    base = lax.dot_general(pc, l8, (((1,), (0,)), ((), ())),
                           preferred_element_type=jnp.float32)

    p0s, p1s = [], []
    for b in range(nb):
        o0b = oh0[b * _SUB:(b + 1) * _SUB, :]
        o1b = oh1[b * _SUB:(b + 1) * _SUB, :]
        v0 = jnp.sum(o0b * (base + cb0s[b] + cw0s[b]), axis=1)
        v1 = jnp.sum(o1b * (base + cnt0 + cb1s[b] + cw1s[b]), axis=1)
        p0s.append(v0.reshape(1, _SUB))
        p1s.append(v1.reshape(1, _SUB))
    p0 = jnp.concatenate(p0s, axis=0)
    p1 = jnp.concatenate(p1s, axis=0)
    pos0_ref[...] = jnp.clip(p0, 0.0, CAP - 1).astype(jnp.int32)
    pos1_ref[...] = jnp.clip(p1, 0.0, CAP - 1).astype(jnp.int32)

    ends = base + pc
    nbv = lax.broadcasted_iota(jnp.int32, (1, 128), 1).astype(jnp.float32) * float(TB)
    acc = jnp.zeros((1, 128), jnp.int32)
    for e in range(NE):
        se = lax.slice(ends, (0, e), (1, e + 1))
        acc = acc + (nbv >= jnp.broadcast_to(se, (1, 128))).astype(jnp.int32)
    be_ref[...] = jnp.minimum(acc, NE - 1)


@jax.jit
def _routing(logits):
    nb = SEQ // _SUB
    return pl.pallas_call(
        _routing_body,
        grid=(1,),
        in_specs=[pl.BlockSpec((SEQ, NE), lambda i: (0, 0))],
        out_specs=[
            pl.BlockSpec((SEQ, TOPK), lambda i: (0, 0)),
            pl.BlockSpec((SEQ, 16), lambda i: (0, 0)),
            pl.BlockSpec((SEQ, 16), lambda i: (0, 0)),
            pl.BlockSpec((nb, _SUB), lambda i: (0, 0)),
            pl.BlockSpec((nb, _SUB), lambda i: (0, 0)),
            pl.BlockSpec((1, 128), lambda i: (0, 0)),
        ],
        out_shape=[
            jax.ShapeDtypeStruct((SEQ, TOPK), jnp.int32),
            jax.ShapeDtypeStruct((SEQ, 16), jnp.float32),
            jax.ShapeDtypeStruct((SEQ, 16), jnp.float32),
            jax.ShapeDtypeStruct((nb, _SUB), jnp.int32),
            jax.ShapeDtypeStruct((nb, _SUB), jnp.int32),
            jax.ShapeDtypeStruct((1, 128), jnp.int32),
        ],
    )(logits)


@functools.lru_cache(maxsize=1)
def _sc_mesh():
    return plsc.VectorSubcoreMesh(core_axis_name="c", subcore_axis_name="s")


@jax.jit
def _dispatch(x, pos0, pos1):
    @functools.partial(
        pl.kernel,
        out_type=jax.ShapeDtypeStruct((CAP, DIM), jnp.float32),
        mesh=_sc_mesh(),
        scratch_types=[
            pltpu.VMEM((TPW,), jnp.int32),
            pltpu.VMEM((TPW,), jnp.int32),
            pltpu.VMEM((TPW, DIM), jnp.float32),
            pltpu.SemaphoreType.DMA,
        ],
    )
    def k(x_hbm, p0_hbm, p1_hbm, xs_hbm, idx0_v, idx1_v, x_v, sem):
        wid = lax.axis_index("s") * 2 + lax.axis_index("c")
        base = wid * TPW
        row = wid // 4
        col = (wid % 4) * TPW
        pltpu.sync_copy(x_hbm.at[pl.ds(base, TPW)], x_v)
        pltpu.sync_copy(p0_hbm.at[row, pl.ds(col, TPW)], idx0_v)
        pltpu.sync_copy(p1_hbm.at[row, pl.ds(col, TPW)], idx1_v)
        c0 = pltpu.async_copy(x_v, xs_hbm.at[idx0_v], sem)
        c1 = pltpu.async_copy(x_v, xs_hbm.at[idx1_v], sem)
        c0.wait()
        c1.wait()

    return k(x, pos0, pos1)


def _ffn_body(be_ref, x_ref, gp_ref, up_ref, dp_ref, o_ref):
    i = pl.program_id(0)
    e = be_ref[0, i]
    xb = x_ref[...].astype(jnp.bfloat16)
    gpe = gp_ref[pl.ds(e, 1), :, :][0].astype(jnp.bfloat16)
    upe = up_ref[pl.ds(e, 1), :, :][0].astype(jnp.bfloat16)
    dpe = dp_ref[pl.ds(e, 1), :, :][0].astype(jnp.bfloat16)
    g = lax.dot_general(xb, gpe, (((1,), (1,)), ((), ())),
                        preferred_element_type=jnp.float32)
    u = lax.dot_general(xb, upe, (((1,), (1,)), ((), ())),
                        preferred_element_type=jnp.float32)
    h = (g * jax.nn.sigmoid(g)) * u
    d = lax.dot_general(h.astype(jnp.bfloat16), dpe,
                        (((1,), (1,)), ((), ())),
                        preferred_element_type=jnp.float32)
    o_ref[...] = d.astype(jnp.bfloat16).astype(jnp.float32)


@jax.jit
def _ffn(be, xs, gp, up, dp):
    grid_spec = pltpu.PrefetchScalarGridSpec(
        num_scalar_prefetch=1,
        grid=(NBLK,),
        in_specs=[
            pl.BlockSpec((TB, DIM), lambda i, be_ref: (i, 0)),
            pl.BlockSpec((NE, DFF, DIM), lambda i, be_ref: (0, 0, 0)),
            pl.BlockSpec((NE, DFF, DIM), lambda i, be_ref: (0, 0, 0)),
            pl.BlockSpec((NE, DIM, DFF), lambda i, be_ref: (0, 0, 0)),
        ],
        out_specs=pl.BlockSpec((TB, DIM), lambda i, be_ref: (i, 0)),
    )
    return pl.pallas_call(
        _ffn_body,
        grid_spec=grid_spec,
        out_shape=jax.ShapeDtypeStruct((CAP, DIM), jnp.float32),
        compiler_params=pltpu.CompilerParams(
            dimension_semantics=("arbitrary",),
        ),
    )(be, xs, gp, up, dp)


@jax.jit
def _combine(h, pos0, pos1, rws0, rws1):
    @functools.partial(
        pl.kernel,
        out_type=jax.ShapeDtypeStruct((SEQ, DIM), jnp.float32),
        mesh=_sc_mesh(),
        scratch_types=[
            pltpu.VMEM((TPW,), jnp.int32),
            pltpu.VMEM((TPW,), jnp.int32),
            pltpu.VMEM((TPW, 16), jnp.float32),
            pltpu.VMEM((TPW, 16), jnp.float32),
            pltpu.VMEM((TPW, DIM), jnp.float32),
            pltpu.VMEM((TPW, DIM), jnp.float32),
            pltpu.SemaphoreType.DMA,
        ],
    )
    def k(h_hbm, p0_hbm, p1_hbm, w0_hbm, w1_hbm, out_hbm,
          idx0_v, idx1_v, w0_v, w1_v, h0_v, h1_v, sem):
        wid = lax.axis_index("s") * 2 + lax.axis_index("c")
        base = wid * TPW
        row = wid // 4
        col = (wid % 4) * TPW
        pltpu.sync_copy(p0_hbm.at[row, pl.ds(col, TPW)], idx0_v)
        pltpu.sync_copy(p1_hbm.at[row, pl.ds(col, TPW)], idx1_v)
        pltpu.sync_copy(w0_hbm.at[pl.ds(base, TPW)], w0_v)
        pltpu.sync_copy(w1_hbm.at[pl.ds(base, TPW)], w1_v)
        c0 = pltpu.async_copy(h_hbm.at[idx0_v], h0_v, sem)
        c1 = pltpu.async_copy(h_hbm.at[idx1_v], h1_v, sem)
        c0.wait()
        c1.wait()

        def body(j, _):
            w0 = w0_v[j, :]
            w1 = w1_v[j, :]
            for c in range(DIM // 16):
                sl = pl.ds(c * 16, 16)
                h0_v[j, sl] = w0 * h0_v[j, sl] + w1 * h1_v[j, sl]
            return 0

        lax.fori_loop(0, TPW, body, 0)
        pltpu.sync_copy(h0_v, out_hbm.at[pl.ds(base, TPW)])

    return k(h, pos0, pos1, rws0, rws1)


def kernel(hidden_states, gate_weight, gate_proj_weight, up_proj_weight, down_proj_weight):
    router_logits = hidden_states.astype(jnp.float32) @ gate_weight.astype(jnp.float32).T
    sel, rws0, rws1, pos0, pos1, be = _routing(router_logits)
    xs = _dispatch(hidden_states, pos0, pos1)
    h = _ffn(be, xs, gate_proj_weight, up_proj_weight, down_proj_weight)
    out = _combine(h, pos0, pos1, rws0, rws1)
    return (out, router_logits, sel)


# speedup vs baseline: 1.1678x; 1.1678x over previous
import functools

import jax
import jax.numpy as jnp
from jax import lax
from jax.experimental import pallas as pl
from jax.experimental.pallas import tpu as pltpu
from jax.experimental.pallas import tpu_sc as plsc

SEQ = 2048
DIM = 768
DFF = 512
NE = 8
TOPK = 2
TB = 256
NBLK = 24
CAP = NBLK * TB
NW = 32
TPW = SEQ // NW
_SUB = 256


def _routing_body(logits_ref, sel_ref, rws0_ref, rws1_ref,
                  pos0_ref, pos1_ref, be_ref):
    l = logits_ref[...]
    ii = lax.broadcasted_iota(jnp.int32, (SEQ, NE), 1)
    m0 = jnp.max(l, axis=1, keepdims=True)
    e0 = jnp.min(jnp.where(l == m0, ii, NE), axis=1, keepdims=True)
    lmask = jnp.where(ii == e0, -jnp.inf, l)
    m1 = jnp.max(lmask, axis=1, keepdims=True)
    e1 = jnp.min(jnp.where(lmask == m1, ii, NE), axis=1, keepdims=True)
    b = jnp.exp(m1 - m0)
    rw0 = 1.0 / (1.0 + b)
    rw1 = b / (1.0 + b)
    sel_ref[...] = jnp.concatenate([e0, e1], axis=1)
    rws0_ref[...] = jnp.broadcast_to(rw0, (SEQ, 16))
    rws1_ref[...] = jnp.broadcast_to(rw1, (SEQ, 16))

    oh0 = (jnp.broadcast_to(e0, (SEQ, NE)) == ii).astype(jnp.float32)
    oh1 = (jnp.broadcast_to(e1, (SEQ, NE)) == ii).astype(jnp.float32)
    nb = SEQ // _SUB
    r_i = lax.broadcasted_iota(jnp.int32, (_SUB, _SUB), 0)
    c_i = lax.broadcasted_iota(jnp.int32, (_SUB, _SUB), 1)
    ltri = (c_i < r_i).astype(jnp.float32)
    cw0s, cw1s, cb0s, cb1s = [], [], [], []
    cb0 = jnp.zeros((1, NE), jnp.float32)
    cb1 = jnp.zeros((1, NE), jnp.float32)
    for b in range(nb):
        o0b = oh0[b * _SUB:(b + 1) * _SUB, :]
        o1b = oh1[b * _SUB:(b + 1) * _SUB, :]
        cw0s.append(lax.dot_general(ltri, o0b, (((1,), (0,)), ((), ())),
                                    preferred_element_type=jnp.float32))
        cw1s.append(lax.dot_general(ltri, o1b, (((1,), (0,)), ((), ())),
                                    preferred_element_type=jnp.float32))
        cb0s.append(cb0)
        cb1s.append(cb1)
        cb0 = cb0 + jnp.sum(o0b, axis=0, keepdims=True)
        cb1 = cb1 + jnp.sum(o1b, axis=0, keepdims=True)
    cnt0 = cb0
    cnt = cb0 + cb1
    pc = jnp.ceil(cnt * (1.0 / TB)) * TB
    r8 = lax.broadcasted_iota(jnp.int32, (NE, NE), 0)
    c8 = lax.broadcasted_iota(jnp.int32, (NE, NE), 1)
    l8 = (r8 < c8).astype(jnp.float32)
    base = lax.dot_general(pc, l8, (((1,), (0,)), ((), ())),
                           preferred_element_type=jnp.float32)

    p0s, p1s = [], []
    for b in range(nb):
        o0b = oh0[b * _SUB:(b + 1) * _SUB, :]
        o1b = oh1[b * _SUB:(b + 1) * _SUB, :]
        v0 = jnp.sum(o0b * (base + cb0s[b] + cw0s[b]), axis=1)
        v1 = jnp.sum(o1b * (base + cnt0 + cb1s[b] + cw1s[b]), axis=1)
        p0s.append(v0.reshape(1, _SUB))
        p1s.append(v1.reshape(1, _SUB))
    p0 = jnp.concatenate(p0s, axis=0)
    p1 = jnp.concatenate(p1s, axis=0)
    pos0_ref[...] = jnp.clip(p0, 0.0, CAP - 1).astype(jnp.int32)
    pos1_ref[...] = jnp.clip(p1, 0.0, CAP - 1).astype(jnp.int32)

    ends = base + pc
    nbv = lax.broadcasted_iota(jnp.int32, (1, 128), 1).astype(jnp.float32) * float(TB)
    acc = jnp.zeros((1, 128), jnp.int32)
    for e in range(NE):
        se = lax.slice(ends, (0, e), (1, e + 1))
        acc = acc + (nbv >= jnp.broadcast_to(se, (1, 128))).astype(jnp.int32)
    be_ref[...] = jnp.minimum(acc, NE - 1)


@jax.jit
def _routing(logits):
    nb = SEQ // _SUB
    return pl.pallas_call(
        _routing_body,
        grid=(1,),
        in_specs=[pl.BlockSpec((SEQ, NE), lambda i: (0, 0))],
        out_specs=[
            pl.BlockSpec((SEQ, TOPK), lambda i: (0, 0)),
            pl.BlockSpec((SEQ, 16), lambda i: (0, 0)),
            pl.BlockSpec((SEQ, 16), lambda i: (0, 0)),
            pl.BlockSpec((nb, _SUB), lambda i: (0, 0)),
            pl.BlockSpec((nb, _SUB), lambda i: (0, 0)),
            pl.BlockSpec((1, 128), lambda i: (0, 0)),
        ],
        out_shape=[
            jax.ShapeDtypeStruct((SEQ, TOPK), jnp.int32),
            jax.ShapeDtypeStruct((SEQ, 16), jnp.float32),
            jax.ShapeDtypeStruct((SEQ, 16), jnp.float32),
            jax.ShapeDtypeStruct((nb, _SUB), jnp.int32),
            jax.ShapeDtypeStruct((nb, _SUB), jnp.int32),
            jax.ShapeDtypeStruct((1, 128), jnp.int32),
        ],
    )(logits)


@functools.lru_cache(maxsize=1)
def _sc_mesh():
    return plsc.VectorSubcoreMesh(core_axis_name="c", subcore_axis_name="s")


@jax.jit
def _dispatch(x, pos0, pos1):
    @functools.partial(
        pl.kernel,
        out_type=jax.ShapeDtypeStruct((CAP, DIM), jnp.float32),
        mesh=_sc_mesh(),
        scratch_types=[
            pltpu.VMEM((TPW,), jnp.int32),
            pltpu.VMEM((TPW,), jnp.int32),
            pltpu.VMEM((TPW, DIM), jnp.float32),
            pltpu.SemaphoreType.DMA,
        ],
    )
    def k(x_hbm, p0_hbm, p1_hbm, xs_hbm, idx0_v, idx1_v, x_v, sem):
        wid = lax.axis_index("s") * 2 + lax.axis_index("c")
        base = wid * TPW
        row = wid // 4
        col = (wid % 4) * TPW
        pltpu.sync_copy(x_hbm.at[pl.ds(base, TPW)], x_v)
        pltpu.sync_copy(p0_hbm.at[row, pl.ds(col, TPW)], idx0_v)
        pltpu.sync_copy(p1_hbm.at[row, pl.ds(col, TPW)], idx1_v)
        c0 = pltpu.async_copy(x_v, xs_hbm.at[idx0_v], sem)
        c1 = pltpu.async_copy(x_v, xs_hbm.at[idx1_v], sem)
        c0.wait()
        c1.wait()

    return k(x, pos0, pos1)


def _ffn_body(be_ref, x_ref, gp_ref, up_ref, dp_ref, o_ref):
    i = pl.program_id(0)
    e = be_ref[0, i]
    xb = x_ref[...].astype(jnp.bfloat16)
    gpe = gp_ref[pl.ds(e, 1), :, :][0].astype(jnp.bfloat16)
    upe = up_ref[pl.ds(e, 1), :, :][0].astype(jnp.bfloat16)
    dpe = dp_ref[pl.ds(e, 1), :, :][0].astype(jnp.bfloat16)
    g = lax.dot_general(xb, gpe, (((1,), (1,)), ((), ())),
                        preferred_element_type=jnp.float32)
    u = lax.dot_general(xb, upe, (((1,), (1,)), ((), ())),
                        preferred_element_type=jnp.float32)
    h = (g * jax.nn.sigmoid(g)) * u
    d = lax.dot_general(h.astype(jnp.bfloat16), dpe,
                        (((1,), (1,)), ((), ())),
                        preferred_element_type=jnp.float32)
    o_ref[...] = d.astype(jnp.bfloat16).astype(jnp.float32)


@jax.jit
def _ffn(be, xs, gp, up, dp):
    grid_spec = pltpu.PrefetchScalarGridSpec(
        num_scalar_prefetch=1,
        grid=(NBLK,),
        in_specs=[
            pl.BlockSpec((TB, DIM), lambda i, be_ref: (i, 0)),
            pl.BlockSpec((NE, DFF, DIM), lambda i, be_ref: (0, 0, 0)),
            pl.BlockSpec((NE, DFF, DIM), lambda i, be_ref: (0, 0, 0)),
            pl.BlockSpec((NE, DIM, DFF), lambda i, be_ref: (0, 0, 0)),
        ],
        out_specs=pl.BlockSpec((TB, DIM), lambda i, be_ref: (i, 0)),
    )
    return pl.pallas_call(
        _ffn_body,
        grid_spec=grid_spec,
        out_shape=jax.ShapeDtypeStruct((CAP, DIM), jnp.float32),
        compiler_params=pltpu.CompilerParams(
            dimension_semantics=("arbitrary",),
        ),
    )(be, xs, gp, up, dp)


@jax.jit
def _combine(h, pos0, pos1, rws0, rws1):
    @functools.partial(
        pl.kernel,
        out_type=jax.ShapeDtypeStruct((SEQ, DIM), jnp.float32),
        mesh=_sc_mesh(),
        scratch_types=[
            pltpu.VMEM((TPW,), jnp.int32),
            pltpu.VMEM((TPW,), jnp.int32),
            pltpu.VMEM((TPW, 16), jnp.float32),
            pltpu.VMEM((TPW, 16), jnp.float32),
            pltpu.VMEM((TPW, DIM), jnp.float32),
            pltpu.VMEM((TPW, DIM), jnp.float32),
            pltpu.SemaphoreType.DMA,
        ],
    )
    def k(h_hbm, p0_hbm, p1_hbm, w0_hbm, w1_hbm, out_hbm,
          idx0_v, idx1_v, w0_v, w1_v, h0_v, h1_v, sem):
        wid = lax.axis_index("s") * 2 + lax.axis_index("c")
        base = wid * TPW
        row = wid // 4
        col = (wid % 4) * TPW
        pltpu.sync_copy(p0_hbm.at[row, pl.ds(col, TPW)], idx0_v)
        pltpu.sync_copy(p1_hbm.at[row, pl.ds(col, TPW)], idx1_v)
        pltpu.sync_copy(w0_hbm.at[pl.ds(base, TPW)], w0_v)
        pltpu.sync_copy(w1_hbm.at[pl.ds(base, TPW)], w1_v)
        c0 = pltpu.async_copy(h_hbm.at[idx0_v], h0_v, sem)
        c1 = pltpu.async_copy(h_hbm.at[idx1_v], h1_v, sem)
        c0.wait()
        c1.wait()

        def body(j, _):
            w0 = w0_v[j, :]
            w1 = w1_v[j, :]
            for c in range(DIM // 16):
                sl = pl.ds(c * 16, 16)
                h0_v[j, sl] = w0 * h0_v[j, sl] + w1 * h1_v[j, sl]
            return 0

        lax.fori_loop(0, TPW, body, 0)
        pltpu.sync_copy(h0_v, out_hbm.at[pl.ds(base, TPW)])

    return k(h, pos0, pos1, rws0, rws1)


def kernel(hidden_states, gate_weight, gate_proj_weight, up_proj_weight, down_proj_weight):
    router_logits = hidden_states.astype(jnp.float32) @ gate_weight.astype(jnp.float32).T
    sel, rws0, rws1, pos0, pos1, be = _routing(router_logits)
    xs = _dispatch(hidden_states, pos0, pos1)
    h = _ffn(be, xs, gate_proj_weight, up_proj_weight, down_proj_weight)
    out = _combine(h, pos0, pos1, rws0, rws1)
    return (out, router_logits, sel)
